# baseline (speedup 1.0000x reference)
import functools

import jax
import jax.numpy as jnp
from jax import lax
from jax.experimental import pallas as pl
from jax.experimental.pallas import tpu as pltpu
from jax.experimental.pallas import tpu_sc as plsc

DIM = 64
BATCH = 16384
GRP = 8
N_GRP = 125000

_NC = 2
_NS = 16
_NW = _NC * _NS
_B_PER_W = BATCH // _NW
_QW = _B_PER_W // 4

_mesh = plsc.VectorSubcoreMesh(core_axis_name="c", subcore_axis_name="s")


@functools.partial(
    pl.kernel,
    mesh=_mesh,
    out_type=jax.ShapeDtypeStruct((DIM, BATCH), jnp.float32),
    scratch_types=[
        pltpu.VMEM((_B_PER_W,), jnp.int32),
        pltpu.VMEM((_B_PER_W, DIM), jnp.float32),
        pltpu.VMEM((DIM, _B_PER_W), jnp.float32),
        pltpu.SemaphoreType.DMA((4,)),
        pltpu.SemaphoreType.DMA,
    ],
    compiler_params=pltpu.CompilerParams(needs_layout_passes=False),
)
def _gather_rows(idx_hbm, tab_hbm, out_hbm, idx_v, sel_v, selt_v, sems, osem):
    wid = lax.axis_index("s") * _NC + lax.axis_index("c")
    base = wid * _B_PER_W

    def make_issue(q):
        def issue_body(g, _):
            vec = idx_v[pl.ds(g * 16, 16)]
            gv = lax.shift_right_logical(vec, 3)
            sv = jnp.bitwise_and(vec, 7)
            for i in range(16):
                gi = lax.squeeze(lax.slice(gv, (i,), (i + 1,)), (0,))
                si = lax.squeeze(lax.slice(sv, (i,), (i + 1,)), (0,))
                pltpu.async_copy(
                    tab_hbm.at[gi, si], sel_v.at[g * 16 + i], sems.at[q]
                )
            return ()

        return issue_body

    pltpu.sync_copy(idx_hbm.at[pl.ds(base, _B_PER_W)], idx_v)
    for q in range(4):
        lax.fori_loop(
            q * (_QW // 16), (q + 1) * (_QW // 16), make_issue(q), (),
            unroll=False,
        )

    lane = lax.iota(jnp.int32, 16)

    def tr_body(b, _):
        rows = lane + b * 16
        for d in range(DIM):
            cols = jnp.bitwise_and(lane + d, DIM - 1)
            vals = plsc.load_gather(sel_v, [rows, cols])
            plsc.store_scatter(selt_v, [cols, rows], vals)
        return ()

    for q in range(4):
        for w in range(_QW // GRP):
            pltpu.make_async_copy(
                tab_hbm.at[0],
                sel_v.at[pl.ds(q * _QW + w * GRP, GRP), :],
                sems.at[q],
            ).wait()
        lax.fori_loop(q * (_QW // 16), (q + 1) * (_QW // 16), tr_body, (),
                      unroll=False)
        pltpu.async_copy(
            selt_v.at[:, pl.ds(q * _QW, _QW)],
            out_hbm.at[:, pl.ds(base + q * _QW, _QW)],
            osem,
        )
    for q in range(4):
        pltpu.make_async_copy(
            selt_v.at[:, pl.ds(0, _QW)],
            out_hbm.at[:, pl.ds(0, _QW)],
            osem,
        ).wait()


def kernel(ind, z):
    if ind.ndim == 0:
        ind = ind.reshape((1,))
    z3 = z.reshape(N_GRP, GRP, DIM)
    out_t = _gather_rows(ind, z3)
    return out_t.T.reshape(ind.shape[0], 1, DIM)


# speedup vs baseline: 1.0634x; 1.0634x over previous
import functools

import jax
import jax.numpy as jnp
from jax import lax
from jax.experimental import pallas as pl
from jax.experimental.pallas import tpu as pltpu
from jax.experimental.pallas import tpu_sc as plsc

DIM = 64
BATCH = 16384
GRP = 8
N_GRP = 125000

_NC = 2
_NS = 16
_NW = _NC * _NS
_B_PER_W = BATCH // _NW
_VECS = _B_PER_W // 16

_mesh = plsc.VectorSubcoreMesh(core_axis_name="c", subcore_axis_name="s")


@functools.partial(
    pl.kernel,
    mesh=_mesh,
    out_type=jax.ShapeDtypeStruct((BATCH, DIM), jnp.float32),
    scratch_types=[
        pltpu.VMEM((_B_PER_W,), jnp.int32),
        pltpu.VMEM((_B_PER_W, DIM), jnp.float32),
        pltpu.SemaphoreType.DMA((4,)),
        pltpu.SemaphoreType.DMA,
    ],
    compiler_params=pltpu.CompilerParams(needs_layout_passes=False),
)
def _gather_rows(idx_hbm, tab_hbm, out_hbm, idx_v, sel_v, sems, osem):
    wid = lax.axis_index("s") * _NC + lax.axis_index("c")
    base = wid * _B_PER_W
    _QW = _B_PER_W // 4

    def make_issue(q):
        def issue_body(g, _):
            vec = idx_v[pl.ds(g * 16, 16)]
            gv = lax.shift_right_logical(vec, 3)
            sv = jnp.bitwise_and(vec, 7)
            for i in range(16):
                gi = lax.squeeze(lax.slice(gv, (i,), (i + 1,)), (0,))
                si = lax.squeeze(lax.slice(sv, (i,), (i + 1,)), (0,))
                pltpu.async_copy(
                    tab_hbm.at[gi, si], sel_v.at[g * 16 + i], sems.at[q]
                )
            return ()

        return issue_body

    pltpu.sync_copy(idx_hbm.at[pl.ds(base, _B_PER_W)], idx_v)
    for q in range(4):
        lax.fori_loop(
            q * (_QW // 16), (q + 1) * (_QW // 16), make_issue(q), (),
            unroll=False,
        )
    for q in range(4):
        pltpu.make_async_copy(
            out_hbm.at[pl.ds(0, _QW), :],
            sel_v.at[pl.ds(q * _QW, _QW), :],
            sems.at[q],
        ).wait()
        pltpu.async_copy(
            sel_v.at[pl.ds(q * _QW, _QW), :],
            out_hbm.at[pl.ds(base + q * _QW, _QW), :],
            osem,
        )
    for q in range(4):
        pltpu.make_async_copy(
            out_hbm.at[pl.ds(0, _QW), :],
            sel_v.at[pl.ds(0, _QW), :],
            osem,
        ).wait()


def kernel(ind, z):
    if ind.ndim == 0:
        ind = ind.reshape((1,))
    z3 = z.reshape(N_GRP, GRP, DIM)
    out = _gather_rows(ind, z3)
    return out.reshape(ind.shape[0], 1, DIM)
